# baseline (speedup 1.0000x reference)
import functools

import jax
import jax.numpy as jnp
from jax import lax
from jax.experimental import pallas as pl
from jax.experimental.pallas import tpu as pltpu
from jax.experimental.pallas import tpu_sc as plsc

D = 300
DP = 304
NUM_DEPEND = 100000
NUM_SIBLINGS = 100000
SB_START = NUM_DEPEND + 2
P = 1024 * 20
K = 10
NW = 32
PPW = P // NW
C = 8
CK = C * K
NCHUNK = PPW // C
OFFS = list(range(0, DP, 16))


def _body(re_hbm, sb_hbm, idx_hbm, w_hbm, out_hbm,
          idxv, reiv, sbiv, rows, sbrows, wv, obuf, sem_a, sem_b):
    wid = lax.axis_index("s") * 2 + lax.axis_index("c")
    pltpu.sync_copy(w_hbm, wv)

    def chunk(c, carry):
        ibase = wid * (PPW * K) + c * CK
        pltpu.sync_copy(idx_hbm.at[pl.ds(ibase, CK)], idxv)
        zero = jnp.zeros((16,), jnp.int32)
        sb_off = jnp.full((16,), SB_START - 1, jnp.int32)
        for i in range(CK // 16):
            v = idxv[pl.ds(i * 16, 16)]
            is_sb = v >= SB_START
            reiv[pl.ds(i * 16, 16)] = jnp.where(is_sb, zero, v)
            sbiv[pl.ds(i * 16, 16)] = jnp.where(is_sb, v - sb_off, zero)
        ca = pltpu.async_copy(re_hbm.at[reiv], rows, sem_a)
        cb = pltpu.async_copy(sb_hbm.at[sbiv], sbrows, sem_b)
        ca.wait()
        cb.wait()
        for p in range(C):
            r0 = p * K
            accs = []
            for off in OFFS:
                a = rows[r0, pl.ds(off, 16)]
                for k in range(1, K):
                    a = a + rows[r0 + k, pl.ds(off, 16)]
                accs.append(a)
            s = sbrows[r0, :]
            for k in range(1, K):
                s = s + sbrows[r0 + k, :]
            accs[0] = accs[0] + s
            for j, off in enumerate(OFFS):
                obuf[p, pl.ds(off, 16)] = accs[j] * wv[pl.ds(off, 16)]
        obase = wid * PPW + c * C
        pltpu.sync_copy(obuf, out_hbm.at[pl.ds(obase, C), :])
        return carry

    lax.fori_loop(0, NCHUNK, chunk, 0)


_sc_call = functools.partial(
    pl.kernel,
    mesh=plsc.VectorSubcoreMesh(core_axis_name="c", subcore_axis_name="s"),
    compiler_params=pltpu.CompilerParams(use_tc_tiling_on_sc=False),
    out_type=jax.ShapeDtypeStruct((P, DP), jnp.float32),
    scratch_types=[
        pltpu.VMEM((CK,), jnp.int32),
        pltpu.VMEM((CK,), jnp.int32),
        pltpu.VMEM((CK,), jnp.int32),
        pltpu.VMEM((CK, DP), jnp.float32),
        pltpu.VMEM((CK, 16), jnp.float32),
        pltpu.VMEM((DP,), jnp.float32),
        pltpu.VMEM((C, DP), jnp.float32),
        pltpu.SemaphoreType.DMA,
        pltpu.SemaphoreType.DMA,
    ],
)(_body)


def kernel(inputs, re_lut, sb_lut, w):
    emb_re = jnp.zeros((NUM_DEPEND + 2, DP), jnp.float32)
    emb_re = emb_re.at[1:, :D].set(re_lut)
    sb16 = jnp.zeros((NUM_SIBLINGS + 2, 16), jnp.float32)
    sb16 = sb16.at[1:, :15].set(sb_lut)
    wk = jnp.zeros((DP,), jnp.float32).at[:D].set((w * (1.0 / K)).reshape(D))
    out = _sc_call(emb_re, sb16, inputs.reshape(P * K), wk)
    return out[:, :D].reshape(1024, 20, D)


# speedup vs baseline: 1.0004x; 1.0004x over previous
import functools

import jax
import jax.numpy as jnp
from jax import lax
from jax.experimental import pallas as pl
from jax.experimental.pallas import tpu as pltpu
from jax.experimental.pallas import tpu_sc as plsc

D = 300
DP = 304
NUM_DEPEND = 100000
NUM_SIBLINGS = 100000
SB_START = NUM_DEPEND + 2
P = 1024 * 20
K = 10
NW = 32
PPW = P // NW
C = 8
CK = C * K
NCHUNK = PPW // C
NBUF = 4
GROUPS = NCHUNK // NBUF
OFFS = list(range(0, DP, 16))


def _body(re_hbm, sb_hbm, idx_hbm, w_hbm, out_hbm, *scr):
    raw = scr[0:4]
    rei = scr[4:8]
    sbi = scr[8:12]
    rows = scr[12:16]
    sbr = scr[16:20]
    wv = scr[20]
    obuf = scr[21]
    sems_a = scr[22:26]
    sems_b = scr[26:30]

    wid = lax.axis_index("s") * 2 + lax.axis_index("c")
    widx = wid * (PPW * K)
    pltpu.sync_copy(w_hbm, wv)

    zero = jnp.zeros((16,), jnp.int32)
    sb_off = jnp.full((16,), SB_START - 1, jnp.int32)

    def issue(c, b):
        pltpu.sync_copy(idx_hbm.at[pl.ds(widx + c * CK, CK)], raw[b])
        for i in range(CK // 16):
            v = raw[b][pl.ds(i * 16, 16)]
            is_sb = v >= SB_START
            rei[b][pl.ds(i * 16, 16)] = jnp.where(is_sb, zero, v)
            sbi[b][pl.ds(i * 16, 16)] = jnp.where(is_sb, v - sb_off, zero)
        pltpu.async_copy(re_hbm.at[rei[b]], rows[b], sems_a[b])
        pltpu.async_copy(sb_hbm.at[sbi[b]], sbr[b], sems_b[b])

    for b in range(NBUF - 1):
        issue(jnp.int32(b), b)

    def group(g, carry):
        for b in range(NBUF):
            c = g * NBUF + b
            pltpu.make_async_copy(re_hbm.at[rei[b]], rows[b], sems_a[b]).wait()
            pltpu.make_async_copy(sb_hbm.at[sbi[b]], sbr[b], sems_b[b]).wait()

            def accum(p, cc, _b=b):
                r0 = p * K
                accs = []
                for off in OFFS:
                    a = rows[_b][r0, pl.ds(off, 16)]
                    for k in range(1, K):
                        a = a + rows[_b][r0 + k, pl.ds(off, 16)]
                    accs.append(a)
                s = sbr[_b][r0, :]
                for k in range(1, K):
                    s = s + sbr[_b][r0 + k, :]
                accs[0] = accs[0] + s
                orow = _b * C + p
                for j, off in enumerate(OFFS):
                    obuf[orow, pl.ds(off, 16)] = accs[j] * wv[pl.ds(off, 16)]
                return cc

            lax.fori_loop(0, C, accum, 0)

            @pl.when(c + NBUF - 1 < NCHUNK)
            def _(c=c, b=b):
                issue(c + NBUF - 1, (b + NBUF - 1) % NBUF)

        obase = wid * PPW + g * (NBUF * C)
        pltpu.sync_copy(obuf, out_hbm.at[pl.ds(obase, NBUF * C), :])
        return carry

    lax.fori_loop(0, GROUPS, group, 0)


_sc_call = functools.partial(
    pl.kernel,
    mesh=plsc.VectorSubcoreMesh(core_axis_name="c", subcore_axis_name="s"),
    compiler_params=pltpu.CompilerParams(use_tc_tiling_on_sc=False),
    out_type=jax.ShapeDtypeStruct((P, DP), jnp.float32),
    scratch_types=(
        [pltpu.VMEM((CK,), jnp.int32) for _ in range(4)]
        + [pltpu.VMEM((CK,), jnp.int32) for _ in range(4)]
        + [pltpu.VMEM((CK,), jnp.int32) for _ in range(4)]
        + [pltpu.VMEM((CK, DP), jnp.float32) for _ in range(4)]
        + [pltpu.VMEM((CK, 16), jnp.float32) for _ in range(4)]
        + [pltpu.VMEM((DP,), jnp.float32)]
        + [pltpu.VMEM((NBUF * C, DP), jnp.float32)]
        + [pltpu.SemaphoreType.DMA for _ in range(8)]
    ),
)(_body)


def kernel(inputs, re_lut, sb_lut, w):
    emb_re = jnp.zeros((NUM_DEPEND + 2, DP), jnp.float32)
    emb_re = emb_re.at[1:, :D].set(re_lut)
    sb16 = jnp.zeros((NUM_SIBLINGS + 2, 16), jnp.float32)
    sb16 = sb16.at[1:, :15].set(sb_lut)
    wk = jnp.zeros((DP,), jnp.float32).at[:D].set((w * (1.0 / K)).reshape(D))
    out = _sc_call(emb_re, sb16, inputs.reshape(P * K), wk)
    return out[:, :D].reshape(1024, 20, D)
